# baseline (speedup 1.0000x reference)
import functools

import jax
import jax.numpy as jnp
from jax import lax
from jax.experimental import pallas as pl
from jax.experimental.pallas import tpu as pltpu
from jax.experimental.pallas import tpu_sc as plsc

NC, NS, L = 2, 16, 16
NW = NC * NS
D = 128
B = 16384 * 200
BPW = B // NW
CH = 400
NCHUNK = BPW // CH

_mesh = plsc.VectorSubcoreMesh(
    core_axis_name="c", subcore_axis_name="s", num_cores=NC, num_subcores=NS
)


@functools.partial(
    pl.kernel,
    out_type=jax.ShapeDtypeStruct((B, D), jnp.float32),
    mesh=_mesh,
    scratch_types=[
        pltpu.VMEM((7, D), jnp.float32),
        pltpu.VMEM((CH,), jnp.int32),
        pltpu.VMEM((CH,), jnp.int32),
        pltpu.VMEM((CH, D), jnp.float32),
        pltpu.VMEM((CH, D), jnp.float32),
        pltpu.SemaphoreType.DMA,
        pltpu.SemaphoreType.DMA,
        pltpu.SemaphoreType.DMA,
        pltpu.SemaphoreType.DMA,
    ],
    compiler_params=pltpu.CompilerParams(needs_layout_passes=False),
)
def _emb(d_hbm, table_hbm, out_hbm, table_v, idx0, idx1, rows0, rows1,
         isem0, isem1, osem0, osem1):
    idx = (idx0, idx1)
    rows = (rows0, rows1)
    isem = (isem0, isem1)
    osem = (osem0, osem1)
    wid = lax.axis_index("s") * NC + lax.axis_index("c")
    base = wid * BPW
    pltpu.sync_copy(table_hbm, table_v)
    lane = lax.iota(jnp.int32, L)

    def start_idx(g, b):
        pltpu.async_copy(d_hbm.at[pl.ds(base + g * CH, CH)], idx[b], isem[b])

    def wait_idx(b):
        pltpu.make_async_copy(
            d_hbm.at[pl.ds(base, CH)], idx[b], isem[b]
        ).wait()

    def start_out(g, b):
        pltpu.async_copy(rows[b], out_hbm.at[pl.ds(base + g * CH, CH)], osem[b])

    def wait_out(b):
        pltpu.make_async_copy(
            rows[b], out_hbm.at[pl.ds(base, CH)], osem[b]
        ).wait()

    def expand(b):
        idx_b = idx[b]
        rows_b = rows[b]

        @plsc.parallel_loop(0, CH // L, unroll=2)
        def _(r):
            idx16 = idx_b[pl.ds(r * L, L)]
            for i in range(L):
                ksplat = idx16.at[jnp.full((L,), i, jnp.int32)].get(
                    mode="promise_in_bounds"
                )
                row = r * L + i
                for j in range(D // L):
                    rows_b[row, pl.ds(j * L, L)] = plsc.load_gather(
                        table_v, [ksplat, j * L + lane]
                    )

    start_idx(0, 0)
    start_idx(1, 1)

    def pair_body(h, carry):
        for b in range(2):
            g = 2 * h + b
            wait_idx(b)

            @pl.when(h > 0)
            def _():
                wait_out(b)

            expand(b)
            start_out(g, b)

            @pl.when(g + 2 < NCHUNK)
            def _():
                start_idx(g + 2, b)

        return carry

    lax.fori_loop(0, NCHUNK // 2, pair_body, 0)
    wait_out(0)
    wait_out(1)


def kernel(d, table):
    out = _emb(d.reshape(B), table)
    return out.reshape(d.shape[0], d.shape[1], D)


# speedup vs baseline: 2.1952x; 2.1952x over previous
import functools

import jax
import jax.numpy as jnp
from jax import lax
from jax.experimental import pallas as pl
from jax.experimental.pallas import tpu as pltpu
from jax.experimental.pallas import tpu_sc as plsc

NC, NS, L = 2, 16, 16
NW = NC * NS
D = 128
B = 16384 * 200
BPW = B // NW
CH = 400
NCHUNK = BPW // CH

_mesh = plsc.VectorSubcoreMesh(
    core_axis_name="c", subcore_axis_name="s", num_cores=NC, num_subcores=NS
)


@functools.partial(
    pl.kernel,
    out_type=jax.ShapeDtypeStruct((B, D), jnp.float32),
    mesh=_mesh,
    scratch_types=[
        pltpu.VMEM((7, D), jnp.float32),
        pltpu.VMEM((CH,), jnp.int32),
        pltpu.VMEM((CH,), jnp.int32),
        pltpu.VMEM((CH, D), jnp.float32),
        pltpu.VMEM((CH, D), jnp.float32),
        pltpu.SemaphoreType.DMA,
        pltpu.SemaphoreType.DMA,
        pltpu.SemaphoreType.DMA,
        pltpu.SemaphoreType.DMA,
    ],
    compiler_params=pltpu.CompilerParams(needs_layout_passes=False),
)
def _emb(d_hbm, table_hbm, out_hbm, table_v, idx0, idx1, rows0, rows1,
         isem0, isem1, osem0, osem1):
    idx = (idx0, idx1)
    rows = (rows0, rows1)
    isem = (isem0, isem1)
    osem = (osem0, osem1)
    wid = lax.axis_index("s") * NC + lax.axis_index("c")
    base = wid * BPW
    pltpu.sync_copy(table_hbm, table_v)
    lane = lax.iota(jnp.int32, L)

    def start_idx(g, b):
        pltpu.async_copy(d_hbm.at[pl.ds(base + g * CH, CH)], idx[b], isem[b])

    def wait_idx(b):
        pltpu.make_async_copy(
            d_hbm.at[pl.ds(base, CH)], idx[b], isem[b]
        ).wait()

    def start_out(g, b):
        pltpu.async_copy(rows[b], out_hbm.at[pl.ds(base + g * CH, CH)], osem[b])

    def wait_out(b):
        pltpu.make_async_copy(
            rows[b], out_hbm.at[pl.ds(base, CH)], osem[b]
        ).wait()

    def expand(b):
        idx_b = idx[b]
        rows_b = rows[b]

        @plsc.parallel_loop(0, CH, unroll=4)
        def _(i):
            ksplat = plsc.load_gather(idx_b, [jnp.full((L,), i, jnp.int32)])
            for j in range(D // L):
                rows_b[i, pl.ds(j * L, L)] = plsc.load_gather(
                    table_v, [ksplat, j * L + lane]
                )

    start_idx(0, 0)
    start_idx(1, 1)

    def pair_body(h, carry):
        for b in range(2):
            g = 2 * h + b
            wait_idx(b)

            @pl.when(h > 0)
            def _():
                wait_out(b)

            expand(b)
            start_out(g, b)

            @pl.when(g + 2 < NCHUNK)
            def _():
                start_idx(g + 2, b)

        return carry

    lax.fori_loop(0, NCHUNK // 2, pair_body, 0)
    wait_out(0)
    wait_out(1)


def kernel(d, table):
    out = _emb(d.reshape(B), table)
    return out.reshape(d.shape[0], d.shape[1], D)
